# Initial kernel scaffold; baseline (speedup 1.0000x reference)
import jax
import jax.numpy as jnp
from jax.experimental import pallas as pl


def kernel(video_flat):
    raise NotImplementedError("write your pallas kernel here")


# speedup vs baseline: 2.3833x; 2.3833x over previous
import functools

import numpy as np

import jax
import jax.numpy as jnp
from jax.experimental import pallas as pl
from jax.experimental.pallas import tpu as pltpu

_VMEM_LIMIT = 64 * 1024 * 1024


def _interp_matrix_np(out_size: int, in_size: int) -> np.ndarray:
    if in_size == 1:
        return np.ones((out_size, 1), np.float32)
    if out_size == 1:
        pos = np.zeros((1,), np.float64)
    else:
        pos = np.arange(out_size, dtype=np.float64) * (
            (in_size - 1) / (out_size - 1))
    lo = np.clip(np.floor(pos).astype(np.int64), 0, in_size - 2)
    frac = (pos - lo).astype(np.float32)
    m = np.zeros((out_size, in_size), np.float32)
    m[np.arange(out_size), lo] += 1.0 - frac
    m[np.arange(out_size), lo + 1] += frac
    return m


@functools.lru_cache(maxsize=None)
def _weights_np(in_h, in_w, out_h, out_w, tm):
    wy = _interp_matrix_np(out_h, in_h)
    wxt = _interp_matrix_np(out_w, in_w).T
    bwy = np.kron(np.eye(tm, dtype=np.float32), wy)
    return np.ascontiguousarray(bwy), np.ascontiguousarray(wxt)


def _resize_kernel(bwy_ref, wxt_ref, img_ref, out_ref, *, H, W, iH, iW, tm):
    img = img_ref[...].reshape(tm * H, W)
    tmp = jnp.dot(img, wxt_ref[...],
                  preferred_element_type=jnp.float32)
    out = jnp.dot(bwy_ref[...], tmp,
                  preferred_element_type=jnp.float32)
    out_ref[...] = out.reshape(tm, iH, iW)


def kernel(video_flat):
    M, H, W = video_flat.shape
    iH, iW = 24, 32
    tm = 8
    assert M % tm == 0

    bwy_np, wxt_np = _weights_np(H, W, iH, iW, tm)
    bwy = jnp.asarray(bwy_np)
    wxt = jnp.asarray(wxt_np)

    grid = (M // tm,)
    cost = pl.CostEstimate(
        flops=2 * M * H * W * iW + 2 * M * iH * H * iW,
        transcendentals=0,
        bytes_accessed=(M * H * W + M * iH * iW) * 4)
    out = pl.pallas_call(
        functools.partial(_resize_kernel, H=H, W=W, iH=iH, iW=iW, tm=tm),
        out_shape=jax.ShapeDtypeStruct((M, iH, iW), jnp.float32),
        grid=grid,
        in_specs=[
            pl.BlockSpec((tm * iH, tm * H), lambda g: (0, 0)),
            pl.BlockSpec((W, iW), lambda g: (0, 0)),
            pl.BlockSpec((tm, H, W), lambda g: (g, 0, 0)),
        ],
        out_specs=pl.BlockSpec((tm, iH, iW), lambda g: (g, 0, 0)),
        compiler_params=pltpu.CompilerParams(
            dimension_semantics=("parallel",),
            vmem_limit_bytes=_VMEM_LIMIT),
        cost_estimate=cost,
    )(bwy, wxt, video_flat.astype(jnp.float32))
    return out
